# baseline (speedup 1.0000x reference)
import functools

import jax
import jax.numpy as jnp
from jax import lax
from jax.experimental import pallas as pl
from jax.experimental.pallas import tpu as pltpu
from jax.experimental.pallas import tpu_sc as plsc

NC = 2
NS = 16
CHUNK = 96
RING = 4
PF = 2


@functools.lru_cache(maxsize=None)
def _segsum_builder(N, D, C, RPT, ACC_R):
    HD = D // NC
    mesh = plsc.VectorSubcoreMesh(core_axis_name="c", subcore_axis_name="s")

    @functools.partial(
        pl.kernel,
        mesh=mesh,
        compiler_params=pltpu.CompilerParams(use_tc_tiling_on_sc=False),
        out_type=jax.ShapeDtypeStruct((NC, ACC_R, HD), jnp.float32),
        scratch_types=[
            pltpu.VMEM((C, CHUNK), jnp.int32),
            pltpu.VMEM((C, CHUNK), jnp.int32),
        ]
        + [pltpu.VMEM((CHUNK, HD), jnp.float32) for _ in range(RING)]
        + [pltpu.VMEM_SHARED((ACC_R, HD), jnp.float32)]
        + [pltpu.SemaphoreType.DMA for _ in range(2 * RING)],
    )
    def segsum(xs_hbm, src_hbm, dst_hbm, zeros_hbm, out_hbm,
               src_v, dst_v, *refs):
        rows = refs[:RING]
        acc_s = refs[RING]
        gsem = refs[RING + 1:RING + 1 + RING]
        ssem = refs[RING + 1 + RING:]
        c = lax.axis_index("c")
        s = lax.axis_index("s")
        xh = xs_hbm.at[c]
        pltpu.sync_copy(zeros_hbm, acc_s.at[pl.ds(s * RPT, RPT)])
        pltpu.sync_copy(src_hbm.at[s], src_v)
        pltpu.sync_copy(dst_hbm.at[s], dst_v)
        for b in range(PF):
            pltpu.async_copy(xh.at[src_v.at[b]], rows[b], gsem[b])
        plsc.subcore_barrier()

        def group(jj, carry):
            for b in range(RING):
                j = jj * RING + b
                bp = (b + PF) % RING
                @pl.when(j + PF < C)
                def _():
                    pltpu.async_copy(
                        xh.at[src_v.at[j + PF]], rows[bp], gsem[bp])
                pltpu.make_async_copy(
                    xh.at[src_v.at[j]], rows[b], gsem[b]).wait()
                pltpu.sync_copy(rows[b], acc_s.at[dst_v.at[j]], add=True)
            return carry

        lax.fori_loop(0, C // RING, group, 0)
        plsc.subcore_barrier()
        pltpu.sync_copy(acc_s.at[pl.ds(s * RPT, RPT)],
                        out_hbm.at[c, pl.ds(s * RPT, RPT)])

    return segsum


def _mlp_body(x_ref, p_ref, w1_ref, b1_ref, w2_ref, b2_ref, o_ref):
    h = jnp.concatenate(
        [x_ref[0] + p_ref[0], x_ref[1] + p_ref[1]], axis=-1)
    a = jnp.maximum(
        jnp.dot(h, w1_ref[...], preferred_element_type=jnp.float32)
        + b1_ref[...], 0.0)
    o = (jnp.dot(a, w2_ref[...], preferred_element_type=jnp.float32)
         + b2_ref[...])
    hd = o.shape[-1] // 2
    o_ref[0] = o[:, :hd]
    o_ref[1] = o[:, hd:]


def _mlp_last_body(x_ref, p_ref, w1_ref, b1_ref, w2_ref, b2_ref, gs_ref):
    h = jnp.concatenate(
        [x_ref[0] + p_ref[0], x_ref[1] + p_ref[1]], axis=-1)
    a = jnp.maximum(
        jnp.dot(h, w1_ref[...], preferred_element_type=jnp.float32)
        + b1_ref[...], 0.0)
    o = (jnp.dot(a, w2_ref[...], preferred_element_type=jnp.float32)
         + b2_ref[...])
    gs_ref[...] = jnp.sum(o, axis=0, keepdims=True)[None]


def _heads_body(ps_ref, w1a_ref, b1a_ref, v2a_ref, c2a_ref,
                w1b_ref, b1b_ref, v2b_ref, c2b_ref, o_ref):
    g = jnp.sum(ps_ref[...], axis=0)
    za = jnp.maximum(
        jnp.dot(g, w1a_ref[...], preferred_element_type=jnp.float32)
        + b1a_ref[...], 0.0)
    oa = jnp.sum(za * v2a_ref[...], axis=1, keepdims=True) + c2a_ref[...]
    zb = jnp.maximum(
        jnp.dot(g, w1b_ref[...], preferred_element_type=jnp.float32)
        + b1b_ref[...], 0.0)
    ob = jnp.sum(zb * v2b_ref[...], axis=1, keepdims=True) + c2b_ref[...]
    lane = lax.broadcasted_iota(jnp.int32, (1, 128), 1)
    o_ref[...] = jnp.where(lane == 0, oa, jnp.where(lane == 1, ob, 0.0))


def kernel(X, edge_index, edge_weights, params):
    N, D = X.shape
    H = params[2].shape[0]
    E = edge_index.shape[1]
    HD = D // NC
    num_layers = (len(params) - 8) // 4

    C = -(-E // (NS * CHUNK))
    C = -(-C // RING) * RING
    E_pad = NS * C * CHUNK
    RPT = -(-(-(-(N + 1) // NS)) // 8) * 8
    ACC_R = RPT * NS

    src = edge_index[0]
    dst = edge_index[1]
    pad = E_pad - E
    src_p = jnp.concatenate(
        [src, jnp.zeros((pad,), jnp.int32)]).reshape(NS, C, CHUNK)
    dst_p = jnp.concatenate(
        [dst, jnp.full((pad,), N, jnp.int32)]).reshape(NS, C, CHUNK)
    zeros_t = jnp.zeros((RPT, HD), jnp.float32)

    segsum = _segsum_builder(N, D, C, RPT, ACC_R)

    BLK = 2000
    grid = N // BLK
    split_spec = pl.BlockSpec((NC, BLK, HD), lambda i: (0, i, 0))
    full2 = lambda shape: pl.BlockSpec(shape, lambda i: (0, 0))

    mlp = pl.pallas_call(
        _mlp_body,
        grid=(grid,),
        in_specs=[split_spec, split_spec, full2((D, H)), full2((1, H)),
                  full2((H, H)), full2((1, H))],
        out_specs=split_spec,
        out_shape=jax.ShapeDtypeStruct((NC, N, HD), jnp.float32),
    )
    mlp_last = pl.pallas_call(
        _mlp_last_body,
        grid=(grid,),
        in_specs=[split_spec, split_spec, full2((D, H)), full2((1, H)),
                  full2((H, H)), full2((1, H))],
        out_specs=pl.BlockSpec((1, 1, H), lambda i: (i, 0, 0)),
        out_shape=jax.ShapeDtypeStruct((grid, 1, H), jnp.float32),
    )
    heads = pl.pallas_call(
        _heads_body,
        grid=(1,),
        in_specs=[pl.BlockSpec((grid, 1, H), lambda i: (0, 0, 0)),
                  full2((H, H)), full2((1, H)), full2((1, H)), full2((1, 1)),
                  full2((H, H)), full2((1, H)), full2((1, H)), full2((1, 1))],
        out_specs=full2((1, 128)),
        out_shape=jax.ShapeDtypeStruct((1, 128), jnp.float32),
    )

    xs = jnp.stack([X[:, :HD], X[:, HD:]], axis=0)
    for i in range(num_layers):
        W1, b1, W2, b2 = params[4 * i: 4 * i + 4]
        parts = segsum(xs, src_p, dst_p, zeros_t)
        if i < num_layers - 1:
            xs = mlp(xs, parts, W1, b1.reshape(1, H), W2, b2.reshape(1, H))
        else:
            gs = mlp_last(xs, parts, W1, b1.reshape(1, H), W2,
                          b2.reshape(1, H))

    Wo1a, bo1a, Wo2a, bo2a, Wo1b, bo1b, Wo2b, bo2b = params[4 * num_layers:]
    out = heads(gs,
                Wo1a, bo1a.reshape(1, H), Wo2a.reshape(1, H),
                bo2a.reshape(1, 1),
                Wo1b, bo1b.reshape(1, H), Wo2b.reshape(1, H),
                bo2b.reshape(1, 1))
    return out[0, :2]


# speedup vs baseline: 1.3837x; 1.3837x over previous
import functools

import jax
import jax.numpy as jnp
from jax import lax
from jax.experimental import pallas as pl
from jax.experimental.pallas import tpu as pltpu
from jax.experimental.pallas import tpu_sc as plsc

NC = 2
NS = 16
CHUNK = 80
RING = 4
PF = 2


@functools.lru_cache(maxsize=None)
def _segsum_builder(N, D, C, RPT, ACC_R):
    HD = D // NC
    mesh = plsc.VectorSubcoreMesh(core_axis_name="c", subcore_axis_name="s")

    @functools.partial(
        pl.kernel,
        mesh=mesh,
        compiler_params=pltpu.CompilerParams(use_tc_tiling_on_sc=False),
        out_type=jax.ShapeDtypeStruct((NC, ACC_R, HD), jnp.float32),
        scratch_types=[
            pltpu.VMEM((C, CHUNK), jnp.int32),
            pltpu.VMEM((C, CHUNK), jnp.int32),
        ]
        + [pltpu.VMEM((CHUNK, HD), jnp.float32) for _ in range(RING)]
        + [pltpu.VMEM_SHARED((ACC_R, HD), jnp.float32)]
        + [pltpu.SemaphoreType.DMA for _ in range(2 * RING)],
    )
    def segsum(xs_hbm, src_hbm, dst_hbm, zeros_hbm, out_hbm,
               src_v, dst_v, *refs):
        rows = refs[:RING]
        acc_s = refs[RING]
        gsem = refs[RING + 1:RING + 1 + RING]
        ssem = refs[RING + 1 + RING:]
        c = lax.axis_index("c")
        s = lax.axis_index("s")
        xh = xs_hbm.at[c]
        pltpu.sync_copy(zeros_hbm, acc_s.at[pl.ds(s * RPT, RPT)])
        pltpu.sync_copy(src_hbm.at[s], src_v)
        pltpu.sync_copy(dst_hbm.at[s], dst_v)
        for b in range(PF):
            pltpu.async_copy(xh.at[src_v.at[b]], rows[b], gsem[b])
        plsc.subcore_barrier()

        def group(jj, carry):
            for b in range(RING):
                j = jj * RING + b
                bp = (b + PF) % RING
                @pl.when(j + PF < C)
                def _():
                    pltpu.async_copy(
                        xh.at[src_v.at[j + PF]], rows[bp], gsem[bp])
                pltpu.make_async_copy(
                    xh.at[src_v.at[j]], rows[b], gsem[b]).wait()
                pltpu.sync_copy(rows[b], acc_s.at[dst_v.at[j]], add=True)
            return carry

        lax.fori_loop(0, C // RING, group, 0)
        plsc.subcore_barrier()
        pltpu.sync_copy(acc_s.at[pl.ds(s * RPT, RPT)],
                        out_hbm.at[c, pl.ds(s * RPT, RPT)])

    return segsum


def _mlp_body(x_ref, p_ref, w1_ref, b1_ref, w2_ref, b2_ref, o_ref):
    h = jnp.concatenate(
        [x_ref[0] + p_ref[0], x_ref[1] + p_ref[1]], axis=-1)
    a = jnp.maximum(
        jnp.dot(h, w1_ref[...], preferred_element_type=jnp.float32)
        + b1_ref[...], 0.0)
    o = (jnp.dot(a, w2_ref[...], preferred_element_type=jnp.float32)
         + b2_ref[...])
    hd = o.shape[-1] // 2
    o_ref[0] = o[:, :hd]
    o_ref[1] = o[:, hd:]


def _mlp_last_body(x_ref, p_ref, w1_ref, b1_ref, w2_ref, b2_ref, gs_ref):
    h = jnp.concatenate(
        [x_ref[0] + p_ref[0], x_ref[1] + p_ref[1]], axis=-1)
    a = jnp.maximum(
        jnp.dot(h, w1_ref[...], preferred_element_type=jnp.float32)
        + b1_ref[...], 0.0)
    o = (jnp.dot(a, w2_ref[...], preferred_element_type=jnp.float32)
         + b2_ref[...])
    gs_ref[...] = jnp.sum(o, axis=0, keepdims=True)[None]


def _heads_body(ps_ref, w1a_ref, b1a_ref, v2a_ref, c2a_ref,
                w1b_ref, b1b_ref, v2b_ref, c2b_ref, o_ref):
    g = jnp.sum(ps_ref[...], axis=0)
    za = jnp.maximum(
        jnp.dot(g, w1a_ref[...], preferred_element_type=jnp.float32)
        + b1a_ref[...], 0.0)
    oa = jnp.sum(za * v2a_ref[...], axis=1, keepdims=True) + c2a_ref[...]
    zb = jnp.maximum(
        jnp.dot(g, w1b_ref[...], preferred_element_type=jnp.float32)
        + b1b_ref[...], 0.0)
    ob = jnp.sum(zb * v2b_ref[...], axis=1, keepdims=True) + c2b_ref[...]
    lane = lax.broadcasted_iota(jnp.int32, (1, 128), 1)
    o_ref[...] = jnp.where(lane == 0, oa, jnp.where(lane == 1, ob, 0.0))


def kernel(X, edge_index, edge_weights, params):
    N, D = X.shape
    H = params[2].shape[0]
    E = edge_index.shape[1]
    HD = D // NC
    num_layers = (len(params) - 8) // 4

    C = -(-E // (NS * CHUNK))
    C = -(-C // RING) * RING
    E_pad = NS * C * CHUNK
    RPT = -(-(-(-(N + 1) // NS)) // 8) * 8
    ACC_R = RPT * NS

    src = edge_index[0]
    dst = edge_index[1]
    pad = E_pad - E
    src_p = jnp.concatenate(
        [src, jnp.zeros((pad,), jnp.int32)]).reshape(NS, C, CHUNK)
    dst_p = jnp.concatenate(
        [dst, jnp.full((pad,), N, jnp.int32)]).reshape(NS, C, CHUNK)
    zeros_t = jnp.zeros((RPT, HD), jnp.float32)

    segsum = _segsum_builder(N, D, C, RPT, ACC_R)

    BLK = 2000
    grid = N // BLK
    split_spec = pl.BlockSpec((NC, BLK, HD), lambda i: (0, i, 0))
    full2 = lambda shape: pl.BlockSpec(shape, lambda i: (0, 0))

    mlp = pl.pallas_call(
        _mlp_body,
        grid=(grid,),
        in_specs=[split_spec, split_spec, full2((D, H)), full2((1, H)),
                  full2((H, H)), full2((1, H))],
        out_specs=split_spec,
        out_shape=jax.ShapeDtypeStruct((NC, N, HD), jnp.float32),
    )
    mlp_last = pl.pallas_call(
        _mlp_last_body,
        grid=(grid,),
        in_specs=[split_spec, split_spec, full2((D, H)), full2((1, H)),
                  full2((H, H)), full2((1, H))],
        out_specs=pl.BlockSpec((1, 1, H), lambda i: (i, 0, 0)),
        out_shape=jax.ShapeDtypeStruct((grid, 1, H), jnp.float32),
    )
    heads = pl.pallas_call(
        _heads_body,
        grid=(1,),
        in_specs=[pl.BlockSpec((grid, 1, H), lambda i: (0, 0, 0)),
                  full2((H, H)), full2((1, H)), full2((1, H)), full2((1, 1)),
                  full2((H, H)), full2((1, H)), full2((1, H)), full2((1, 1))],
        out_specs=full2((1, 128)),
        out_shape=jax.ShapeDtypeStruct((1, 128), jnp.float32),
    )

    xs = jnp.stack([X[:, :HD], X[:, HD:]], axis=0)
    for i in range(num_layers):
        W1, b1, W2, b2 = params[4 * i: 4 * i + 4]
        parts = segsum(xs, src_p, dst_p, zeros_t)
        if i < num_layers - 1:
            xs = mlp(xs, parts, W1, b1.reshape(1, H), W2, b2.reshape(1, H))
        else:
            gs = mlp_last(xs, parts, W1, b1.reshape(1, H), W2,
                          b2.reshape(1, H))

    Wo1a, bo1a, Wo2a, bo2a, Wo1b, bo1b, Wo2b, bo2b = params[4 * num_layers:]
    out = heads(gs,
                Wo1a, bo1a.reshape(1, H), Wo2a.reshape(1, H),
                bo2a.reshape(1, 1),
                Wo1b, bo1b.reshape(1, H), Wo2b.reshape(1, H),
                bo2b.reshape(1, 1))
    return out[0, :2]


# speedup vs baseline: 1.7480x; 1.2633x over previous
import functools

import jax
import jax.numpy as jnp
from jax import lax
from jax.experimental import pallas as pl
from jax.experimental.pallas import tpu as pltpu
from jax.experimental.pallas import tpu_sc as plsc

NC = 2
NS = 16
CHUNK = 88
RING = 4
PF = 2


@functools.lru_cache(maxsize=None)
def _segsum_builder(N, D, C, RPT, ACC_R):
    HD = D // NC
    mesh = plsc.VectorSubcoreMesh(core_axis_name="c", subcore_axis_name="s")

    @functools.partial(
        pl.kernel,
        mesh=mesh,
        compiler_params=pltpu.CompilerParams(use_tc_tiling_on_sc=False),
        out_type=jax.ShapeDtypeStruct((NC, ACC_R, HD), jnp.float32),
        scratch_types=[
            pltpu.VMEM((C, CHUNK), jnp.int32),
            pltpu.VMEM((C, CHUNK), jnp.int32),
        ]
        + [pltpu.VMEM((CHUNK, HD), jnp.float32) for _ in range(RING)]
        + [pltpu.VMEM_SHARED((ACC_R, HD), jnp.float32)]
        + [pltpu.SemaphoreType.DMA for _ in range(2 * RING)],
    )
    def segsum(xs_hbm, src_hbm, dst_hbm, zeros_hbm, out_hbm,
               src_v, dst_v, *refs):
        rows = refs[:RING]
        acc_s = refs[RING]
        gsem = refs[RING + 1:RING + 1 + RING]
        ssem = refs[RING + 1 + RING:]
        c = lax.axis_index("c")
        s = lax.axis_index("s")
        xh = xs_hbm.at[c]
        pltpu.sync_copy(zeros_hbm, acc_s.at[pl.ds(s * RPT, RPT)])
        pltpu.sync_copy(src_hbm.at[s], src_v)
        pltpu.sync_copy(dst_hbm.at[s], dst_v)
        for b in range(PF):
            pltpu.async_copy(xh.at[src_v.at[b]], rows[b], gsem[b])
        plsc.subcore_barrier()

        def group(jj, carry):
            for b in range(RING):
                j = jj * RING + b
                bp = (b + PF) % RING
                @pl.when(j + PF < C)
                def _():
                    pltpu.async_copy(
                        xh.at[src_v.at[j + PF]], rows[bp], gsem[bp])
                pltpu.make_async_copy(
                    xh.at[src_v.at[j]], rows[b], gsem[b]).wait()
                pltpu.sync_copy(rows[b], acc_s.at[dst_v.at[j]], add=True)
            return carry

        lax.fori_loop(0, C // RING, group, 0)
        plsc.subcore_barrier()
        pltpu.sync_copy(acc_s.at[pl.ds(s * RPT, RPT)],
                        out_hbm.at[c, pl.ds(s * RPT, RPT)])

    return segsum


def _mlp_body(x_ref, p_ref, w1_ref, b1_ref, w2_ref, b2_ref, o_ref):
    h = jnp.concatenate(
        [x_ref[0] + p_ref[0], x_ref[1] + p_ref[1]], axis=-1)
    a = jnp.maximum(
        jnp.dot(h, w1_ref[...], preferred_element_type=jnp.float32)
        + b1_ref[...], 0.0)
    o = (jnp.dot(a, w2_ref[...], preferred_element_type=jnp.float32)
         + b2_ref[...])
    hd = o.shape[-1] // 2
    o_ref[0] = o[:, :hd]
    o_ref[1] = o[:, hd:]


def _mlp_last_body(x_ref, p_ref, w1_ref, b1_ref, w2_ref, b2_ref, gs_ref):
    h = jnp.concatenate(
        [x_ref[0] + p_ref[0], x_ref[1] + p_ref[1]], axis=-1)
    a = jnp.maximum(
        jnp.dot(h, w1_ref[...], preferred_element_type=jnp.float32)
        + b1_ref[...], 0.0)
    o = (jnp.dot(a, w2_ref[...], preferred_element_type=jnp.float32)
         + b2_ref[...])
    gs_ref[...] = jnp.sum(o, axis=0, keepdims=True)[None]


def _heads_body(ps_ref, w1a_ref, b1a_ref, v2a_ref, c2a_ref,
                w1b_ref, b1b_ref, v2b_ref, c2b_ref, o_ref):
    g = jnp.sum(ps_ref[...], axis=0)
    za = jnp.maximum(
        jnp.dot(g, w1a_ref[...], preferred_element_type=jnp.float32)
        + b1a_ref[...], 0.0)
    oa = jnp.sum(za * v2a_ref[...], axis=1, keepdims=True) + c2a_ref[...]
    zb = jnp.maximum(
        jnp.dot(g, w1b_ref[...], preferred_element_type=jnp.float32)
        + b1b_ref[...], 0.0)
    ob = jnp.sum(zb * v2b_ref[...], axis=1, keepdims=True) + c2b_ref[...]
    lane = lax.broadcasted_iota(jnp.int32, (1, 128), 1)
    o_ref[...] = jnp.where(lane == 0, oa, jnp.where(lane == 1, ob, 0.0))


def kernel(X, edge_index, edge_weights, params):
    N, D = X.shape
    H = params[2].shape[0]
    E = edge_index.shape[1]
    HD = D // NC
    num_layers = (len(params) - 8) // 4

    C = -(-E // (NS * CHUNK))
    C = -(-C // RING) * RING
    E_pad = NS * C * CHUNK
    RPT = -(-(-(-(N + 1) // NS)) // 8) * 8
    ACC_R = RPT * NS

    src = edge_index[0]
    dst = edge_index[1]
    pad = E_pad - E
    src_p = jnp.concatenate(
        [src, jnp.zeros((pad,), jnp.int32)]).reshape(NS, C, CHUNK)
    dst_p = jnp.concatenate(
        [dst, jnp.full((pad,), N, jnp.int32)]).reshape(NS, C, CHUNK)
    zeros_t = jnp.zeros((RPT, HD), jnp.float32)

    segsum = _segsum_builder(N, D, C, RPT, ACC_R)

    BLK = 2000
    grid = N // BLK
    split_spec = pl.BlockSpec((NC, BLK, HD), lambda i: (0, i, 0))
    full2 = lambda shape: pl.BlockSpec(shape, lambda i: (0, 0))

    mlp = pl.pallas_call(
        _mlp_body,
        grid=(grid,),
        in_specs=[split_spec, split_spec, full2((D, H)), full2((1, H)),
                  full2((H, H)), full2((1, H))],
        out_specs=split_spec,
        out_shape=jax.ShapeDtypeStruct((NC, N, HD), jnp.float32),
    )
    mlp_last = pl.pallas_call(
        _mlp_last_body,
        grid=(grid,),
        in_specs=[split_spec, split_spec, full2((D, H)), full2((1, H)),
                  full2((H, H)), full2((1, H))],
        out_specs=pl.BlockSpec((1, 1, H), lambda i: (i, 0, 0)),
        out_shape=jax.ShapeDtypeStruct((grid, 1, H), jnp.float32),
    )
    heads = pl.pallas_call(
        _heads_body,
        grid=(1,),
        in_specs=[pl.BlockSpec((grid, 1, H), lambda i: (0, 0, 0)),
                  full2((H, H)), full2((1, H)), full2((1, H)), full2((1, 1)),
                  full2((H, H)), full2((1, H)), full2((1, H)), full2((1, 1))],
        out_specs=full2((1, 128)),
        out_shape=jax.ShapeDtypeStruct((1, 128), jnp.float32),
    )

    xs = jnp.stack([X[:, :HD], X[:, HD:]], axis=0)
    for i in range(num_layers):
        W1, b1, W2, b2 = params[4 * i: 4 * i + 4]
        parts = segsum(xs, src_p, dst_p, zeros_t)
        if i < num_layers - 1:
            xs = mlp(xs, parts, W1, b1.reshape(1, H), W2, b2.reshape(1, H))
        else:
            gs = mlp_last(xs, parts, W1, b1.reshape(1, H), W2,
                          b2.reshape(1, H))

    Wo1a, bo1a, Wo2a, bo2a, Wo1b, bo1b, Wo2b, bo2b = params[4 * num_layers:]
    out = heads(gs,
                Wo1a, bo1a.reshape(1, H), Wo2a.reshape(1, H),
                bo2a.reshape(1, 1),
                Wo1b, bo1b.reshape(1, H), Wo2b.reshape(1, H),
                bo2b.reshape(1, 1))
    return out[0, :2]
